# baseline (speedup 1.0000x reference)
import functools

import jax
import jax.numpy as jnp
from jax import lax
from jax.experimental import pallas as pl
from jax.experimental.pallas import tpu as pltpu
from jax.experimental.pallas import tpu_sc as plsc

_IN_CH = 128
_OUT_CH = 128
_BATCH = 4096
_W_COLS = _IN_CH * _OUT_CH
_LANE = 128

_NC = 2
_NS = 16
_NW = _NC * _NS
_BPW = _BATCH // _NW

_NQ = 4
_QB = _BPW // _NQ
_SUB = 4
_NT = _IN_CH // _SUB
_NSTEP = _NQ * _NT
_R = 6
_NTAIL = _NSTEP % _R
_NGROUP = _NSTEP // _R


@jax.jit
def _lookup(table, idx, idx4):
    mesh = plsc.VectorSubcoreMesh(core_axis_name="c", subcore_axis_name="s")

    @functools.partial(
        pl.kernel,
        mesh=mesh,
        out_type=(
            jax.ShapeDtypeStruct((_BATCH, _OUT_CH, _IN_CH), jnp.float32),
            jax.ShapeDtypeStruct((_BATCH, _OUT_CH), jnp.float32),
        ),
        scratch_types=[
            pltpu.VMEM((_BPW,), jnp.int32),
            pltpu.VMEM((_NQ, _QB), jnp.int32),
            pltpu.VMEM((_R, _QB, _SUB, _LANE), jnp.float32),
            pltpu.VMEM((_BPW, _OUT_CH), jnp.float32),
            pltpu.SemaphoreType.DMA,
            pltpu.SemaphoreType.DMA,
            pltpu.SemaphoreType.DMA,
            pltpu.SemaphoreType.DMA,
            pltpu.SemaphoreType.DMA,
            pltpu.SemaphoreType.DMA,
            pltpu.SemaphoreType.DMA,
            pltpu.SemaphoreType.DMA,
            pltpu.SemaphoreType.DMA,
            pltpu.SemaphoreType.DMA,
            pltpu.SemaphoreType.DMA,
            pltpu.SemaphoreType.DMA,
            pltpu.SemaphoreType.DMA,
        ],
    )
    def k(table_hbm, idx_hbm, idx4_hbm, w_hbm, b_hbm,
          idx_v, idx4_v, bufs, bias_v,
          sb, sg0, sg1, sg2, sg3, sg4, sg5, so0, so1, so2, so3, so4, so5):
        sg = (sg0, sg1, sg2, sg3, sg4, sg5)
        so = (so0, so1, so2, so3, so4, so5)
        wid = lax.axis_index("s") * _NC + lax.axis_index("c")
        base = wid * _BPW
        pltpu.sync_copy(idx_hbm.at[wid], idx_v)
        pltpu.sync_copy(idx4_hbm.at[wid], idx4_v)

        bias_cp = pltpu.async_copy(
            table_hbm.at[idx_v, pl.ds(_W_COLS, _OUT_CH)], bias_v, sb)

        def gather_step(s, slot):
            q = s // _NT
            t = s % _NT
            for kk in range(_SUB):
                pltpu.async_copy(
                    table_hbm.at[idx4_v.at[q],
                                 pl.ds((t * _SUB + kk) * _LANE, _LANE)],
                    bufs.at[slot, :, kk, :], sg[slot])

        def wait_gather_step(slot):
            for kk in range(_SUB):
                pltpu.make_async_copy(
                    table_hbm.at[idx4_v.at[0], pl.ds(0, _LANE)],
                    bufs.at[slot, :, kk, :], sg[slot]).wait()

        def write_step(s, slot):
            q = s // _NT
            t = s % _NT
            return pltpu.async_copy(
                bufs.at[slot],
                w_hbm.at[pl.ds(base + q * _QB, _QB),
                         pl.ds(t * _SUB, _SUB), :], so[slot])

        for b in range(_R):
            gather_step(b, b)

        def body(g, carry):
            outs = []
            for b in range(_R):
                wait_gather_step(b)
                outs.append(write_step(g * _R + b, b))
            for b in range(_R):
                outs[b].wait()

                @pl.when((g + 1) * _R + b < _NSTEP)
                def _():
                    gather_step((g + 1) * _R + b, b)

            return carry

        lax.fori_loop(0, _NGROUP, body, 0)

        tail_outs = []
        for b in range(_NTAIL):
            wait_gather_step(b)
            tail_outs.append(write_step(_NGROUP * _R + b, b))
        for cp in tail_outs:
            cp.wait()

        bias_cp.wait()
        pltpu.sync_copy(bias_v, b_hbm.at[pl.ds(base, _BPW), :])

    return k(table, idx, idx4)


def kernel(table, obj_idx):
    idx = obj_idx.astype(jnp.int32).reshape(_NW, _BPW)
    w, b_flat = _lookup(table, idx, idx.reshape(_NW, _NQ, _QB))
    return w, b_flat.reshape(_BATCH, 1, _OUT_CH)


# speedup vs baseline: 1.0051x; 1.0051x over previous
import functools

import jax
import jax.numpy as jnp
from jax import lax
from jax.experimental import pallas as pl
from jax.experimental.pallas import tpu as pltpu
from jax.experimental.pallas import tpu_sc as plsc

_IN_CH = 128
_OUT_CH = 128
_BATCH = 4096
_W_COLS = _IN_CH * _OUT_CH
_LANE = 128

_NC = 2
_NS = 16
_NW = _NC * _NS
_BPW = _BATCH // _NW

_NQ = 4
_QB = _BPW // _NQ
_SUB = 8
_NT = _IN_CH // _SUB
_NSTEP = _NQ * _NT
_R = 3
_NGROUP = (_NSTEP - 1) // _R


@jax.jit
def _lookup(table, idx, idx4):
    mesh = plsc.VectorSubcoreMesh(core_axis_name="c", subcore_axis_name="s")

    @functools.partial(
        pl.kernel,
        mesh=mesh,
        out_type=(
            jax.ShapeDtypeStruct((_BATCH, _OUT_CH, _IN_CH), jnp.float32),
            jax.ShapeDtypeStruct((_BATCH, _OUT_CH), jnp.float32),
        ),
        scratch_types=[
            pltpu.VMEM((_BPW,), jnp.int32),
            pltpu.VMEM((_NQ, _QB), jnp.int32),
            pltpu.VMEM((_QB, _SUB, _LANE), jnp.float32),
            pltpu.VMEM((_QB, _SUB, _LANE), jnp.float32),
            pltpu.VMEM((_QB, _SUB, _LANE), jnp.float32),
            pltpu.VMEM((_BPW, _OUT_CH), jnp.float32),
            pltpu.SemaphoreType.DMA,
            pltpu.SemaphoreType.DMA,
            pltpu.SemaphoreType.DMA,
            pltpu.SemaphoreType.DMA,
            pltpu.SemaphoreType.DMA,
            pltpu.SemaphoreType.DMA,
            pltpu.SemaphoreType.DMA,
        ],
    )
    def k(table_hbm, idx_hbm, idx4_hbm, w_hbm, b_hbm,
          idx_v, idx4_v, buf0, buf1, buf2, bias_v,
          sb, sg0, sg1, sg2, so0, so1, so2):
        bufs = (buf0, buf1, buf2)
        sg = (sg0, sg1, sg2)
        so = (so0, so1, so2)
        wid = lax.axis_index("s") * _NC + lax.axis_index("c")
        base = wid * _BPW
        pltpu.sync_copy(idx_hbm.at[wid], idx_v)
        pltpu.sync_copy(idx4_hbm.at[wid], idx4_v)

        bias_cp = pltpu.async_copy(
            table_hbm.at[idx_v, pl.ds(_W_COLS, _OUT_CH)], bias_v, sb)

        def gather_step(s, slot):
            q = s // _NT
            t = s % _NT
            for kk in range(_SUB):
                pltpu.async_copy(
                    table_hbm.at[idx4_v.at[q],
                                 pl.ds((t * _SUB + kk) * _LANE, _LANE)],
                    bufs[slot].at[:, kk, :], sg[slot])

        def wait_gather_step(slot):
            for kk in range(_SUB):
                pltpu.make_async_copy(
                    table_hbm.at[idx4_v.at[0], pl.ds(0, _LANE)],
                    bufs[slot].at[:, kk, :], sg[slot]).wait()

        def write_step(s, slot):
            q = s // _NT
            t = s % _NT
            return pltpu.async_copy(
                bufs[slot],
                w_hbm.at[pl.ds(base + q * _QB, _QB),
                         pl.ds(t * _SUB, _SUB), :], so[slot])

        for b in range(_R):
            gather_step(b, b)

        def body(g, carry):
            outs = []
            for b in range(_R):
                wait_gather_step(b)
                outs.append(write_step(g * _R + b, b))
            for b in range(_R):
                outs[b].wait()

                @pl.when((g + 1) * _R + b < _NSTEP)
                def _():
                    gather_step((g + 1) * _R + b, b)

            return carry

        lax.fori_loop(0, _NGROUP, body, 0)

        wait_gather_step(0)
        write_step(_NSTEP - 1, 0).wait()

        bias_cp.wait()
        pltpu.sync_copy(bias_v, b_hbm.at[pl.ds(base, _BPW), :])

    return k(table, idx, idx4)


def kernel(table, obj_idx):
    idx = obj_idx.astype(jnp.int32).reshape(_NW, _BPW)
    w, b_flat = _lookup(table, idx, idx.reshape(_NW, _NQ, _QB))
    return w, b_flat.reshape(_BATCH, 1, _OUT_CH)
